# baseline (speedup 1.0000x reference)
import functools

import jax
import jax.numpy as jnp
from jax import lax
from jax.experimental import pallas as pl
from jax.experimental.pallas import tpu as pltpu
from jax.experimental.pallas import tpu_sc as plsc


def _move_rows(data_t):
    f, n = data_t.shape
    info = plsc.get_sparse_core_info()
    nc = info.num_cores
    cols_per_c = n // nc
    cw = 16384
    rounds = cols_per_c // (2 * cw)
    assert cols_per_c % (2 * cw) == 0

    mesh = plsc.VectorSubcoreMesh(core_axis_name="c", subcore_axis_name="s")

    @functools.partial(
        pl.kernel,
        mesh=mesh,
        out_type=jax.ShapeDtypeStruct((f, n), data_t.dtype),
        scratch_types=[
            pltpu.VMEM_SHARED((f, cw), jnp.float32),
            pltpu.VMEM_SHARED((f, cw), jnp.float32),
            pltpu.SemaphoreType.DMA,
            pltpu.SemaphoreType.DMA,
            pltpu.SemaphoreType.DMA,
            pltpu.SemaphoreType.DMA,
        ],
    )
    def k(d_hbm, o_hbm, buf_a, buf_b, rsem_a, rsem_b, wsem_a, wsem_b):
        cid = lax.axis_index("c")
        sid = lax.axis_index("s")
        c0 = cid * cols_per_c

        @pl.when(sid == 0)
        def _():
            def rd(buf, c, sem):
                pltpu.async_copy(d_hbm.at[:, pl.ds(c0 + (c % cols_per_c), cw)],
                                 buf, sem)

            def wr(buf, c, sem):
                pltpu.async_copy(buf, o_hbm.at[:, pl.ds(c0 + c, cw)], sem)

            def drain(desc_src, desc_dst, sem):
                pltpu.make_async_copy(desc_src, desc_dst, sem).wait()

            rd(buf_a, 0, rsem_a)

            def body(t, carry):
                c = t * 2 * cw
                drain(d_hbm.at[:, pl.ds(c0, cw)], buf_a, rsem_a)
                wr(buf_a, c, wsem_a)
                rd(buf_b, c + cw, rsem_b)
                drain(d_hbm.at[:, pl.ds(c0, cw)], buf_b, rsem_b)
                wr(buf_b, c + cw, wsem_b)
                drain(buf_a, o_hbm.at[:, pl.ds(c0, cw)], wsem_a)
                rd(buf_a, c + 2 * cw, rsem_a)
                drain(buf_b, o_hbm.at[:, pl.ds(c0, cw)], wsem_b)
                return carry

            lax.fori_loop(0, rounds, body, 0)
            drain(d_hbm.at[:, pl.ds(c0, cw)], buf_a, rsem_a)

    return k(data_t)


def kernel(data, partitions, index0, index1):
    del partitions, index0, index1
    return _move_rows(data.T).T


# speedup vs baseline: 1.0665x; 1.0665x over previous
import functools

import jax
import jax.numpy as jnp
from jax import lax
from jax.experimental import pallas as pl
from jax.experimental.pallas import tpu as pltpu
from jax.experimental.pallas import tpu_sc as plsc


def _move_rows(data_t):
    f, n = data_t.shape
    info = plsc.get_sparse_core_info()
    nc, ns = info.num_cores, info.num_subcores
    cols_per_sc = n // nc

    nb = 4
    cw = 512
    tile_cols = 16384
    tile_region = (ns - 1) * tile_cols
    sp_region = cols_per_sc - tile_region
    cw2 = 8192
    chunks_t = tile_cols // cw
    rounds_t = chunks_t // nb
    rounds_s = sp_region // (2 * cw2)
    assert n % nc == 0 and tile_cols % (nb * cw) == 0 and rounds_t >= 2
    assert sp_region % (2 * cw2) == 0 and rounds_s >= 1

    mesh = plsc.VectorSubcoreMesh(core_axis_name="c", subcore_axis_name="s")

    @functools.partial(
        pl.kernel,
        mesh=mesh,
        out_type=jax.ShapeDtypeStruct((f, n), data_t.dtype),
        scratch_types=[
            [pltpu.VMEM((f, cw), data_t.dtype) for _ in range(nb)],
            [pltpu.SemaphoreType.DMA for _ in range(nb)],
            [pltpu.SemaphoreType.DMA for _ in range(nb)],
            pltpu.VMEM_SHARED((f, cw2), jnp.float32),
            pltpu.VMEM_SHARED((f, cw2), jnp.float32),
            pltpu.SemaphoreType.DMA,
            pltpu.SemaphoreType.DMA,
            pltpu.SemaphoreType.DMA,
            pltpu.SemaphoreType.DMA,
        ],
    )
    def k(d_hbm, o_hbm, bufs, rsems, wsems,
          sp_a, sp_b, sp_ra, sp_rb, sp_wa, sp_wb):
        cid = lax.axis_index("c")
        sid = lax.axis_index("s")
        sc_base = cid * cols_per_sc

        @pl.when(sid > 0)
        def _tile_path():
            c0 = sc_base + (sid - 1) * tile_cols

            def rd(b, c):
                pltpu.async_copy(
                    d_hbm.at[:, pl.ds(c0 + (c % chunks_t) * cw, cw)],
                    bufs[b], rsems[b])

            def wr(b, c):
                pltpu.async_copy(bufs[b], o_hbm.at[:, pl.ds(c0 + c * cw, cw)],
                                 wsems[b])

            def drain_r(b):
                pltpu.make_async_copy(d_hbm.at[:, pl.ds(c0, cw)], bufs[b],
                                      rsems[b]).wait()

            def drain_w(b):
                pltpu.make_async_copy(bufs[b], o_hbm.at[:, pl.ds(c0, cw)],
                                      wsems[b]).wait()

            def slot(b, c, first):
                drain_r(b)
                wr(b, c)
                if not first:
                    bp = (b + nb - 1) % nb
                    drain_w(bp)
                    rd(bp, c + nb - 1)

            for b in range(nb - 1):
                rd(b, b)
            slot(0, 0, True)
            rd(nb - 1, nb - 1)
            for b in range(1, nb):
                slot(b, b, False)

            def body(t, carry):
                c = t * nb
                for b in range(nb):
                    slot(b, c + b, False)
                return carry

            lax.fori_loop(1, rounds_t, body, 0)

            drain_w(nb - 1)
            for b in range(nb - 1):
                drain_r(b)

        @pl.when(sid == 0)
        def _spmem_path():
            c0 = sc_base + tile_region

            def rd(buf, c, sem):
                pltpu.async_copy(d_hbm.at[:, pl.ds(c0 + (c % sp_region), cw2)],
                                 buf, sem)

            def wr(buf, c, sem):
                pltpu.async_copy(buf, o_hbm.at[:, pl.ds(c0 + c, cw2)], sem)

            def drain(src, dst, sem):
                pltpu.make_async_copy(src, dst, sem).wait()

            rd(sp_a, 0, sp_ra)

            def body(t, carry):
                c = t * 2 * cw2
                drain(d_hbm.at[:, pl.ds(c0, cw2)], sp_a, sp_ra)
                wr(sp_a, c, sp_wa)
                rd(sp_b, c + cw2, sp_rb)
                drain(d_hbm.at[:, pl.ds(c0, cw2)], sp_b, sp_rb)
                wr(sp_b, c + cw2, sp_wb)
                drain(sp_a, o_hbm.at[:, pl.ds(c0, cw2)], sp_wa)
                rd(sp_a, c + 2 * cw2, sp_ra)
                drain(sp_b, o_hbm.at[:, pl.ds(c0, cw2)], sp_wb)
                return carry

            lax.fori_loop(0, rounds_s, body, 0)
            drain(d_hbm.at[:, pl.ds(c0, cw2)], sp_a, sp_ra)

    return k(data_t)


def kernel(data, partitions, index0, index1):
    del partitions, index0, index1
    return _move_rows(data.T).T


# speedup vs baseline: 1.0960x; 1.0276x over previous
import functools

import jax
import jax.numpy as jnp
from jax import lax
from jax.experimental import pallas as pl
from jax.experimental.pallas import tpu as pltpu
from jax.experimental.pallas import tpu_sc as plsc


def _move_rows(data_t):
    f, n = data_t.shape
    info = plsc.get_sparse_core_info()
    nw = info.num_cores * info.num_subcores
    cols_per_w = n // nw
    assert n % nw == 0

    mesh = plsc.VectorSubcoreMesh(core_axis_name="c", subcore_axis_name="s")

    nb = 4
    cw = 512
    chunks = cols_per_w // cw
    rounds = chunks // nb
    assert cols_per_w % (nb * cw) == 0 and rounds >= 2

    @functools.partial(
        pl.kernel,
        mesh=mesh,
        out_type=jax.ShapeDtypeStruct((f, n), data_t.dtype),
        scratch_types=[
            [pltpu.VMEM((f, cw), data_t.dtype) for _ in range(nb)],
            [pltpu.SemaphoreType.DMA for _ in range(nb)],
            [pltpu.SemaphoreType.DMA for _ in range(nb)],
        ],
    )
    def k(d_hbm, o_hbm, bufs, rsems, wsems):
        wid = lax.axis_index("s") * info.num_cores + lax.axis_index("c")
        c0 = wid * cols_per_w

        def rd(b, c):
            pltpu.async_copy(d_hbm.at[:, pl.ds(c0 + (c % chunks) * cw, cw)],
                             bufs[b], rsems[b])

        def wr(b, c):
            pltpu.async_copy(bufs[b], o_hbm.at[:, pl.ds(c0 + c * cw, cw)],
                             wsems[b])

        def drain_r(b):
            pltpu.make_async_copy(d_hbm.at[:, pl.ds(c0, cw)], bufs[b],
                                  rsems[b]).wait()

        def drain_w(b):
            pltpu.make_async_copy(bufs[b], o_hbm.at[:, pl.ds(c0, cw)],
                                  wsems[b]).wait()

        def slot(b, c, first):
            drain_r(b)
            wr(b, c)
            if not first:
                bp = (b + nb - 1) % nb
                drain_w(bp)
                rd(bp, c + nb - 1)

        for b in range(nb - 1):
            rd(b, b)
        slot(0, 0, True)
        rd(nb - 1, nb - 1)
        for b in range(1, nb):
            slot(b, b, False)

        def body(t, carry):
            c = t * nb
            for b in range(nb):
                slot(b, c + b, False)
            return carry

        lax.fori_loop(1, rounds, body, 0)

        drain_w(nb - 1)
        for b in range(nb - 1):
            drain_r(b)

    return k(data_t)


def kernel(data, partitions, index0, index1):
    del partitions, index0, index1
    return _move_rows(data.T).T
